# baseline (speedup 1.0000x reference)
import functools

import jax
import jax.numpy as jnp
from jax.experimental import pallas as pl

B = 2048
N_UP = 16
N_DOWN = 16
N_EL = 32
D_NODE = 64
D_EDGE = 16
D_STREAM = 32

BB = 16


def _body(e_up_ref, e_dn_ref, nodes_ref,
          Wwu_ref, bwu_ref, Whu_ref, bhu_ref,
          Wwd_ref, bwd_ref, Whd_ref, bhd_ref,
          out_up_ref, out_dn_ref):
    nodes = nodes_ref[...]

    def one_type(e_ref, Ww_ref, bw_ref, Wh_ref, bh_ref, sender_lo):
        nd = nodes[:, sender_lo:sender_lo + 16, :].reshape(BB * 16, D_NODE)
        hx = (jnp.dot(nd, Wh_ref[...], preferred_element_type=jnp.float32)
              + bh_ref[...]) * (1.0 / 16.0)
        e = e_ref[...].reshape(BB * N_EL * 16, D_EDGE)
        we = (jnp.dot(e, Ww_ref[...], preferred_element_type=jnp.float32)
              + bw_ref[...])
        we4 = we.reshape(BB, N_EL, 16, D_STREAM)
        hx4 = hx.reshape(BB, 1, 16, D_STREAM)
        return jnp.sum(we4 * hx4, axis=2)

    out_up_ref[...] = one_type(e_up_ref, Wwu_ref, bwu_ref, Whu_ref, bhu_ref, 0)
    out_dn_ref[...] = one_type(e_dn_ref, Wwd_ref, bwd_ref, Whd_ref, bhd_ref, N_UP)


@jax.jit
def kernel(nodes, edges_up, edges_down,
           W_w_up, b_w_up, W_h_up, b_h_up,
           W_w_down, b_w_down, W_h_down, b_h_down):
    grid = (B // BB,)
    bspec_edges = pl.BlockSpec((BB, N_EL, 16, D_EDGE), lambda i: (i, 0, 0, 0))
    bspec_nodes = pl.BlockSpec((BB, N_EL, D_NODE), lambda i: (i, 0, 0))
    bspec_w = pl.BlockSpec((D_EDGE, D_STREAM), lambda i: (0, 0))
    bspec_h = pl.BlockSpec((D_NODE, D_STREAM), lambda i: (0, 0))
    bspec_b = pl.BlockSpec((1, D_STREAM), lambda i: (0, 0))
    bspec_out = pl.BlockSpec((BB, N_EL, D_STREAM), lambda i: (i, 0, 0))

    out_shape = (
        jax.ShapeDtypeStruct((B, N_EL, D_STREAM), jnp.float32),
        jax.ShapeDtypeStruct((B, N_EL, D_STREAM), jnp.float32),
    )
    out_up, out_dn = pl.pallas_call(
        _body,
        grid=grid,
        in_specs=[bspec_edges, bspec_edges, bspec_nodes,
                  bspec_w, bspec_b, bspec_h, bspec_b,
                  bspec_w, bspec_b, bspec_h, bspec_b],
        out_specs=[bspec_out, bspec_out],
        out_shape=out_shape,
    )(edges_up, edges_down, nodes,
      W_w_up, b_w_up.reshape(1, D_STREAM), W_h_up, b_h_up.reshape(1, D_STREAM),
      W_w_down, b_w_down.reshape(1, D_STREAM), W_h_down, b_h_down.reshape(1, D_STREAM))
    return (out_up, out_dn)


# speedup vs baseline: 3.1463x; 3.1463x over previous
import jax
import jax.numpy as jnp
from jax.experimental import pallas as pl

B = 2048
N_UP = 16
N_EL = 32
D_NODE = 64
D_EDGE = 16
D_STREAM = 32
NS = 16
DJ = NS * D_STREAM

BB = 32


def _body(e_up_ref, e_dn_ref, nodes_ref,
          W2u_ref, bwu_ref, Wh2u_ref, bhu_ref,
          W2d_ref, bwd_ref, Wh2d_ref, bhd_ref,
          out_up_ref, out_dn_ref):
    nodes = nodes_ref[...]

    def one_type(e_ref, W2_ref, bw_ref, Wh2_ref, bh_ref, col_lo):
        hx = (jnp.dot(nodes[:, col_lo:col_lo + NS * D_NODE], Wh2_ref[...],
                      preferred_element_type=jnp.float32)
              + bh_ref[...])
        e2 = e_ref[...].reshape(BB * N_EL, NS * D_EDGE)
        we = (jnp.dot(e2, W2_ref[...], preferred_element_type=jnp.float32)
              + bw_ref[...])
        m = we.reshape(BB, N_EL, DJ) * hx[:, None, :]
        s = m[..., :256] + m[..., 256:]
        s = s[..., :128] + s[..., 128:]
        s = s[..., :64] + s[..., 64:]
        return s[..., :32] + s[..., 32:]

    out_up_ref[...] = one_type(e_up_ref, W2u_ref, bwu_ref, Wh2u_ref, bhu_ref, 0)
    out_dn_ref[...] = one_type(e_dn_ref, W2d_ref, bwd_ref, Wh2d_ref, bhd_ref,
                               N_UP * D_NODE)


def _block_diag_w(w, rows_per_block):
    eye = jnp.eye(NS, dtype=w.dtype)
    t = eye[:, None, :, None] * w[None, :, None, :]
    return t.reshape(NS * rows_per_block, DJ)


@jax.jit
def kernel(nodes, edges_up, edges_down,
           W_w_up, b_w_up, W_h_up, b_h_up,
           W_w_down, b_w_down, W_h_down, b_h_down):
    e2u = edges_up.reshape(B, N_EL, NS * D_EDGE)
    e2d = edges_down.reshape(B, N_EL, NS * D_EDGE)
    nodes2 = nodes.reshape(B, N_EL * D_NODE)

    scale = 1.0 / NS
    W2u = _block_diag_w(W_w_up, D_EDGE)
    W2d = _block_diag_w(W_w_down, D_EDGE)
    Wh2u = _block_diag_w(W_h_up, D_NODE) * scale
    Wh2d = _block_diag_w(W_h_down, D_NODE) * scale
    bwu = jnp.tile(b_w_up, NS)[None]
    bwd = jnp.tile(b_w_down, NS)[None]
    bhu = (jnp.tile(b_h_up, NS) * scale)[None]
    bhd = (jnp.tile(b_h_down, NS) * scale)[None]

    grid = (B // BB,)
    bspec_e = pl.BlockSpec((BB, N_EL, NS * D_EDGE), lambda i: (i, 0, 0))
    bspec_n = pl.BlockSpec((BB, N_EL * D_NODE), lambda i: (i, 0))
    bspec_W2 = pl.BlockSpec((NS * D_EDGE, DJ), lambda i: (0, 0))
    bspec_Wh2 = pl.BlockSpec((NS * D_NODE, DJ), lambda i: (0, 0))
    bspec_b = pl.BlockSpec((1, DJ), lambda i: (0, 0))
    bspec_out = pl.BlockSpec((BB, N_EL, D_STREAM), lambda i: (i, 0, 0))

    out_shape = (
        jax.ShapeDtypeStruct((B, N_EL, D_STREAM), jnp.float32),
        jax.ShapeDtypeStruct((B, N_EL, D_STREAM), jnp.float32),
    )
    out_up, out_dn = pl.pallas_call(
        _body,
        grid=grid,
        in_specs=[bspec_e, bspec_e, bspec_n,
                  bspec_W2, bspec_b, bspec_Wh2, bspec_b,
                  bspec_W2, bspec_b, bspec_Wh2, bspec_b],
        out_specs=[bspec_out, bspec_out],
        out_shape=out_shape,
    )(e2u, e2d, nodes2,
      W2u, bwu, Wh2u, bhu,
      W2d, bwd, Wh2d, bhd)
    return (out_up, out_dn)


# speedup vs baseline: 10.6177x; 3.3746x over previous
import jax
import jax.numpy as jnp
from jax.experimental import pallas as pl

B = 2048
N_UP = 16
N_EL = 32
D_NODE = 64
D_EDGE = 16
D_STREAM = 32
NS = 16
JD = NS * D_STREAM
JE = NS * D_EDGE

BL = 128


def _body(e_up_ref, e_dn_ref, nodes_ref,
          W2u_ref, bwu_ref, WhTu_ref, bhu_ref,
          W2d_ref, bwd_ref, WhTd_ref, bhd_ref,
          out_up_ref, out_dn_ref):

    def one_type(e_ref, W2_ref, bw_ref, WhT_ref, bh_ref, sender_lo, out_ref):
        WhT = WhT_ref[...]
        bh = bh_ref[...]
        W2 = W2_ref[...]
        bw = bw_ref[...]
        hx_js = [
            jnp.dot(WhT, nodes_ref[sender_lo + j], preferred_element_type=jnp.float32) + bh
            for j in range(NS)
        ]
        hxs = jnp.concatenate(hx_js, axis=0)
        for i in range(N_EL):
            e_i = e_ref[i].reshape(JE, BL)
            we_i = (jnp.dot(W2, e_i, preferred_element_type=jnp.float32)
                    + bw)
            m = we_i * hxs
            s = m[:256] + m[256:]
            s = s[:128] + s[128:]
            s = s[:64] + s[64:]
            out_ref[i] = s[:32] + s[32:]

    one_type(e_up_ref, W2u_ref, bwu_ref, WhTu_ref, bhu_ref, 0, out_up_ref)
    one_type(e_dn_ref, W2d_ref, bwd_ref, WhTd_ref, bhd_ref, N_UP, out_dn_ref)


@jax.jit
def kernel(nodes, edges_up, edges_down,
           W_w_up, b_w_up, W_h_up, b_h_up,
           W_w_down, b_w_down, W_h_down, b_h_down):
    et_up = edges_up.transpose(1, 2, 3, 0)
    et_dn = edges_down.transpose(1, 2, 3, 0)
    nt = nodes.transpose(1, 2, 0)

    scale = 1.0 / NS
    eye = jnp.eye(NS, dtype=jnp.float32)

    def w2_block_diag(w):
        t = eye[:, None, :, None] * w.T[None, :, None, :]
        return t.reshape(JD, JE)

    W2u = w2_block_diag(W_w_up)
    W2d = w2_block_diag(W_w_down)
    WhTu = W_h_up.T * scale
    WhTd = W_h_down.T * scale
    bwu = jnp.tile(b_w_up, NS)[:, None]
    bwd = jnp.tile(b_w_down, NS)[:, None]
    bhu = (b_h_up * scale)[:, None]
    bhd = (b_h_down * scale)[:, None]

    grid = (B // BL,)
    bspec_e = pl.BlockSpec((N_EL, NS, D_EDGE, BL), lambda i: (0, 0, 0, i))
    bspec_n = pl.BlockSpec((N_EL, D_NODE, BL), lambda i: (0, 0, i))
    bspec_W2 = pl.BlockSpec((JD, JE), lambda i: (0, 0))
    bspec_WhT = pl.BlockSpec((D_STREAM, D_NODE), lambda i: (0, 0))
    bspec_bw = pl.BlockSpec((JD, 1), lambda i: (0, 0))
    bspec_bh = pl.BlockSpec((D_STREAM, 1), lambda i: (0, 0))
    bspec_out = pl.BlockSpec((N_EL, D_STREAM, BL), lambda i: (0, 0, i))

    out_shape = (
        jax.ShapeDtypeStruct((N_EL, D_STREAM, B), jnp.float32),
        jax.ShapeDtypeStruct((N_EL, D_STREAM, B), jnp.float32),
    )
    out_up_t, out_dn_t = pl.pallas_call(
        _body,
        grid=grid,
        in_specs=[bspec_e, bspec_e, bspec_n,
                  bspec_W2, bspec_bw, bspec_WhT, bspec_bh,
                  bspec_W2, bspec_bw, bspec_WhT, bspec_bh],
        out_specs=[bspec_out, bspec_out],
        out_shape=out_shape,
    )(et_up, et_dn, nt,
      W2u, bwu, WhTu, bhu,
      W2d, bwd, WhTd, bhd)
    return (out_up_t.transpose(2, 0, 1), out_dn_t.transpose(2, 0, 1))


# speedup vs baseline: 13.9574x; 1.3145x over previous
import jax
import jax.numpy as jnp
from jax.experimental import pallas as pl
from jax.experimental.pallas import tpu as pltpu

B = 2048
N_UP = 16
N_EL = 32
D_NODE = 64
D_EDGE = 16
D_STREAM = 32
NS = 16
JD = NS * D_STREAM
JE = NS * D_EDGE

BL = 256


def _body(e_up_ref, e_dn_ref, nodes_ref,
          W2u_ref, bwu_ref, WhTu_ref, bhu_ref,
          W2d_ref, bwd_ref, WhTd_ref, bhd_ref,
          out_up_ref, out_dn_ref):

    def one_type(e_ref, W2_ref, bw_ref, WhT_ref, bh_ref, sender_lo, out_ref):
        WhT = WhT_ref[...]
        bh = bh_ref[...]
        W2 = W2_ref[...]
        bw = bw_ref[...]
        hx_js = [
            jnp.dot(WhT, nodes_ref[sender_lo + j], preferred_element_type=jnp.float32) + bh
            for j in range(NS)
        ]
        hxs = jnp.concatenate(hx_js, axis=0)
        for i in range(N_EL):
            e_i = e_ref[i].reshape(JE, BL)
            we_i = (jnp.dot(W2, e_i, preferred_element_type=jnp.float32)
                    + bw)
            m = we_i * hxs
            s = m[:256] + m[256:]
            s = s[:128] + s[128:]
            s = s[:64] + s[64:]
            out_ref[i] = s[:32] + s[32:]

    one_type(e_up_ref, W2u_ref, bwu_ref, WhTu_ref, bhu_ref, 0, out_up_ref)
    one_type(e_dn_ref, W2d_ref, bwd_ref, WhTd_ref, bhd_ref, N_UP, out_dn_ref)


@jax.jit
def kernel(nodes, edges_up, edges_down,
           W_w_up, b_w_up, W_h_up, b_h_up,
           W_w_down, b_w_down, W_h_down, b_h_down):
    et_up = edges_up.transpose(1, 2, 3, 0)
    et_dn = edges_down.transpose(1, 2, 3, 0)
    nt = nodes.transpose(1, 2, 0)

    scale = 1.0 / NS
    eye = jnp.eye(NS, dtype=jnp.float32)

    def w2_block_diag(w):
        t = eye[:, None, :, None] * w.T[None, :, None, :]
        return t.reshape(JD, JE)

    W2u = w2_block_diag(W_w_up)
    W2d = w2_block_diag(W_w_down)
    WhTu = W_h_up.T * scale
    WhTd = W_h_down.T * scale
    bwu = jnp.tile(b_w_up, NS)[:, None]
    bwd = jnp.tile(b_w_down, NS)[:, None]
    bhu = (b_h_up * scale)[:, None]
    bhd = (b_h_down * scale)[:, None]

    grid = (B // BL,)
    bspec_e = pl.BlockSpec((N_EL, NS, D_EDGE, BL), lambda i: (0, 0, 0, i))
    bspec_n = pl.BlockSpec((N_EL, D_NODE, BL), lambda i: (0, 0, i))
    bspec_W2 = pl.BlockSpec((JD, JE), lambda i: (0, 0))
    bspec_WhT = pl.BlockSpec((D_STREAM, D_NODE), lambda i: (0, 0))
    bspec_bw = pl.BlockSpec((JD, 1), lambda i: (0, 0))
    bspec_bh = pl.BlockSpec((D_STREAM, 1), lambda i: (0, 0))
    bspec_out = pl.BlockSpec((N_EL, D_STREAM, BL), lambda i: (0, 0, i))

    out_shape = (
        jax.ShapeDtypeStruct((N_EL, D_STREAM, B), jnp.float32),
        jax.ShapeDtypeStruct((N_EL, D_STREAM, B), jnp.float32),
    )
    out_up_t, out_dn_t = pl.pallas_call(
        _body,
        grid=grid,
        in_specs=[bspec_e, bspec_e, bspec_n,
                  bspec_W2, bspec_bw, bspec_WhT, bspec_bh,
                  bspec_W2, bspec_bw, bspec_WhT, bspec_bh],
        out_specs=[bspec_out, bspec_out],
        out_shape=out_shape,
        compiler_params=pltpu.CompilerParams(
            dimension_semantics=("arbitrary",),
            vmem_limit_bytes=100 * 1024 * 1024,
        ),
    )(et_up, et_dn, nt,
      W2u, bwu, WhTu, bhu,
      W2d, bwd, WhTd, bhd)
    return (out_up_t.transpose(2, 0, 1), out_dn_t.transpose(2, 0, 1))
